# baseline (speedup 1.0000x reference)
import functools

import jax
import jax.numpy as jnp
from jax import lax
from jax.experimental import pallas as pl
from jax.experimental.pallas import tpu as pltpu
from jax.experimental.pallas import tpu_sc as plsc

T = 2048
D = 768
E = 16
K = 2
A = T * K
TILE = 256
NT = 32
P = NT * TILE
NSUB = 16
LANES = 16


def _router_body(x_ref, Wr_ref, br_ref, i1_ref, i2_ref, g1_ref, g2_ref):
    xt = x_ref[...]
    logits = jnp.dot(xt, Wr_ref[...], preferred_element_type=jnp.float32)
    logits = logits + br_ref[0]
    m = jnp.max(logits, axis=1, keepdims=True)
    ex = jnp.exp(logits - m)
    probs = ex / jnp.sum(ex, axis=1, keepdims=True)
    col = lax.broadcasted_iota(jnp.int32, probs.shape, 1)
    m1 = jnp.max(probs, axis=1, keepdims=True)
    i1 = jnp.min(jnp.where(probs == m1, col, E), axis=1, keepdims=True)
    p2 = jnp.where(col == i1, -1.0, probs)
    m2 = jnp.max(p2, axis=1, keepdims=True)
    i2 = jnp.min(jnp.where(p2 == m2, col, E), axis=1, keepdims=True)
    i1_ref[...] = i1
    i2_ref[...] = i2
    g1_ref[...] = m1
    g2_ref[...] = m2


def _router(xt, Wr, br):
    return pl.pallas_call(
        _router_body,
        grid=(8,),
        in_specs=[
            pl.BlockSpec((TILE, D), lambda i: (i, 0)),
            pl.BlockSpec((D, E), lambda i: (0, 0)),
            pl.BlockSpec((1, E), lambda i: (0, 0)),
        ],
        out_specs=[
            pl.BlockSpec((TILE, 1), lambda i: (i, 0)),
            pl.BlockSpec((TILE, 1), lambda i: (i, 0)),
            pl.BlockSpec((TILE, 1), lambda i: (i, 0)),
            pl.BlockSpec((TILE, 1), lambda i: (i, 0)),
        ],
        out_shape=[
            jax.ShapeDtypeStruct((T, 1), jnp.int32),
            jax.ShapeDtypeStruct((T, 1), jnp.int32),
            jax.ShapeDtypeStruct((T, 1), jnp.float32),
            jax.ShapeDtypeStruct((T, 1), jnp.float32),
        ],
    )(xt, Wr, br.reshape(1, E))


def _shared_body(x_ref, sW1_ref, sb1_ref, sW2_ref, sb2_ref, out_ref):
    xt = x_ref[...]
    sh = jnp.maximum(
        jnp.dot(xt, sW1_ref[...], preferred_element_type=jnp.float32)
        + sb1_ref[0], 0.0)
    out_ref[...] = (
        jnp.dot(sh, sW2_ref[...], preferred_element_type=jnp.float32)
        + sb2_ref[0])


def _shared(xt, sW1, sb1, sW2, sb2):
    return pl.pallas_call(
        _shared_body,
        grid=(8,),
        in_specs=[
            pl.BlockSpec((TILE, D), lambda i: (i, 0)),
            pl.BlockSpec((D, D), lambda i: (0, 0)),
            pl.BlockSpec((1, D), lambda i: (0, 0)),
            pl.BlockSpec((D, D), lambda i: (0, 0)),
            pl.BlockSpec((1, D), lambda i: (0, 0)),
        ],
        out_specs=pl.BlockSpec((TILE, D), lambda i: (i, 0)),
        out_shape=jax.ShapeDtypeStruct((T, D), jnp.float32),
    )(xt, sW1, sb1.reshape(1, D), sW2, sb2.reshape(1, D))


def _dispatch_body(eflat, gflat, x_hbm,
                   xd_hbm, gd_hbm, dest_hbm, meta_hbm,
                   ev_all, gv_buf, hist16, base_v,
                   dest_a, dest_b, meta_v, xbuf, gbuf, sem, sem2):
    s = lax.axis_index("s")
    a0 = s * (A // NSUB)
    t0 = (s % (NSUB // K)) * (A // NSUB)
    lanes = lax.broadcasted_iota(jnp.int32, (LANES,), 0)
    zeros_i = jnp.zeros((LANES,), jnp.int32)
    ones_i = jnp.full((LANES,), 1, jnp.int32)

    pltpu.sync_copy(eflat, ev_all)
    pltpu.sync_copy(gflat.at[pl.ds(a0, 256)], gv_buf)

    hist16[...] = zeros_i

    def acc(i, carry):
        evv = ev_all[pl.ds(16 * i, 16)]
        plsc.addupdate_scatter(hist16, [evv], ones_i)
        return carry

    lax.fori_loop(0, 16 * s, acc, 0)
    prebase = hist16[...]
    lax.fori_loop(16 * s, A // LANES, acc, 0)
    cnt = hist16[...]
    pc = jnp.bitwise_and(cnt + (TILE - 1), -TILE)
    incl = plsc.cumsum(pc)
    pad_off = incl - pc
    base_v[...] = pad_off + prebase

    @pl.when(s == 0)
    def _meta():
        nt_used = jnp.sum(pc // TILE)
        te = incl // TILE
        meta_v[pl.ds(0, 16)] = jnp.full((LANES,), nt_used, jnp.int32)
        for h in range(2):
            acc = zeros_i
            for i in range(16):
                j = 16 * h + i
                c = jnp.sum(jnp.where(te <= j, 1, 0))
                acc = jnp.where(lanes == i, c, acc)
            meta_v[pl.ds(16 + 16 * h, 16)] = jnp.minimum(acc, E - 1)
        pltpu.sync_copy(meta_v, meta_hbm)

    dest_chunks = (dest_a, dest_b)
    for v in range(16):
        evv = ev_all[pl.ds(a0 + 16 * v, 16)]
        rank = zeros_i
        cnts = zeros_i
        for e in range(E):
            m = evv == e
            cs = plsc.cumsum(jnp.where(m, 1, 0))
            rank = jnp.where(m, cs - 1, rank)
            c = jnp.sum(jnp.where(m, 1, 0))
            cnts = cnts + jnp.where(lanes == e, c, 0)
        baseg = plsc.load_gather(base_v, [evv])
        dest_chunks[v // 8][pl.ds(16 * (v % 8), 16)] = baseg + rank
        base_v[...] = base_v[...] + cnts
    pltpu.sync_copy(dest_a, dest_hbm.at[pl.ds(a0, 128)])
    pltpu.sync_copy(dest_b, dest_hbm.at[pl.ds(a0 + 128, 128)])

    for c in range(2):
        pltpu.sync_copy(x_hbm.at[pl.ds(t0 + 128 * c, 128)], xbuf)
        for v8 in range(8):
            gvv = gv_buf[pl.ds(128 * c + 16 * v8, 16)]
            plsc.store_scatter(gbuf, [lanes + 16 * v8, zeros_i], gvv)
        idx = dest_chunks[c]
        cp1 = pltpu.async_copy(xbuf, xd_hbm.at[idx], sem)
        cp2 = pltpu.async_copy(gbuf, gd_hbm.at[idx], sem2)
        cp1.wait()
        cp2.wait()


def _dispatch(eflat, gflat, xt):
    mesh = plsc.VectorSubcoreMesh(
        core_axis_name="c", subcore_axis_name="s", num_cores=1)
    fn = pl.kernel(
        _dispatch_body,
        out_type=[
            jax.ShapeDtypeStruct((P, D), jnp.float32),
            jax.ShapeDtypeStruct((P, 128), jnp.float32),
            jax.ShapeDtypeStruct((A,), jnp.int32),
            jax.ShapeDtypeStruct((48,), jnp.int32),
        ],
        mesh=mesh,
        scratch_types=[
            pltpu.VMEM((A,), jnp.int32),
            pltpu.VMEM((256,), jnp.float32),
            pltpu.VMEM((16,), jnp.int32),
            pltpu.VMEM((16,), jnp.int32),
            pltpu.VMEM((128,), jnp.int32),
            pltpu.VMEM((128,), jnp.int32),
            pltpu.VMEM((48,), jnp.int32),
            pltpu.VMEM((128, D), jnp.float32),
            pltpu.VMEM((128, 128), jnp.float32),
            pltpu.SemaphoreType.DMA,
            pltpu.SemaphoreType.DMA,
        ],
        compiler_params=pltpu.CompilerParams(needs_layout_passes=False),
    )
    return fn(eflat, gflat, xt)


def _ffn_body(meta_ref, xd_ref, gd_ref, rW1_ref, rb1_ref, rW2_ref, rb2_ref,
              out_ref):
    j = pl.program_id(0)

    @pl.when(j < meta_ref[0])
    def _():
        xb = xd_ref[...]
        g = gd_ref[:, 0:1]
        h = jnp.maximum(
            jnp.dot(xb, rW1_ref[0], preferred_element_type=jnp.float32)
            + rb1_ref[0, 0], 0.0)
        out_ref[...] = (
            jnp.dot(h, rW2_ref[0], preferred_element_type=jnp.float32)
            + rb2_ref[0, 0]) * g


def _ffn(meta, xd, gd, rW1, rb1, rW2, rb2):
    def emap(j, m):
        return (m[16 + jnp.minimum(j, m[0] - 1)], 0, 0)

    grid_spec = pltpu.PrefetchScalarGridSpec(
        num_scalar_prefetch=1,
        grid=(NT,),
        in_specs=[
            pl.BlockSpec((TILE, D), lambda j, m: (j, 0)),
            pl.BlockSpec((TILE, 128), lambda j, m: (j, 0)),
            pl.BlockSpec((1, D, D), emap),
            pl.BlockSpec((1, 1, D), emap),
            pl.BlockSpec((1, D, D), emap),
            pl.BlockSpec((1, 1, D), emap),
        ],
        out_specs=pl.BlockSpec((TILE, D), lambda j, m: (j, 0)),
    )
    return pl.pallas_call(
        _ffn_body,
        grid_spec=grid_spec,
        out_shape=jax.ShapeDtypeStruct((P, D), jnp.float32),
        compiler_params=pltpu.CompilerParams(
            dimension_semantics=("arbitrary",)),
    )(meta, xd, gd, rW1, rb1.reshape(E, 1, D), rW2, rb2.reshape(E, 1, D))


def _combine_body(dest_hbm, eo_hbm, shared_hbm, out_hbm,
                  d1v, d2v, e1buf, e2buf, sbuf, sem):
    wid = lax.axis_index("c") * NSUB + lax.axis_index("s")
    t0 = wid * (T // 32)
    for c in range(2):
        toff = t0 + 32 * c
        pltpu.sync_copy(dest_hbm.at[pl.ds(toff, 32)], d1v)
        pltpu.sync_copy(dest_hbm.at[pl.ds(T + toff, 32)], d2v)
        cp1 = pltpu.async_copy(eo_hbm.at[d1v], e1buf, sem)
        cp2 = pltpu.async_copy(eo_hbm.at[d2v], e2buf, sem)
        pltpu.sync_copy(shared_hbm.at[pl.ds(toff, 32)], sbuf)
        cp1.wait()
        cp2.wait()

        def row(r, _):
            for cc in range(D // LANES):
                sl = pl.ds(LANES * cc, LANES)
                sbuf[r, sl] = sbuf[r, sl] + e1buf[r, sl] + e2buf[r, sl]
            return 0

        lax.fori_loop(0, 32, row, 0)
        pltpu.sync_copy(sbuf, out_hbm.at[pl.ds(toff, 32)])


def _combine(dest, eo, shared):
    mesh = plsc.VectorSubcoreMesh(core_axis_name="c", subcore_axis_name="s")
    fn = pl.kernel(
        _combine_body,
        out_type=jax.ShapeDtypeStruct((T, D), jnp.float32),
        mesh=mesh,
        scratch_types=[
            pltpu.VMEM((32,), jnp.int32),
            pltpu.VMEM((32,), jnp.int32),
            pltpu.VMEM((32, D), jnp.float32),
            pltpu.VMEM((32, D), jnp.float32),
            pltpu.VMEM((32, D), jnp.float32),
            pltpu.SemaphoreType.DMA,
        ],
        compiler_params=pltpu.CompilerParams(needs_layout_passes=False),
    )
    return fn(dest, eo, shared)


@jax.jit
def kernel(x, Wr, br, sW1, sb1, sW2, sb2, rW1, rb1, rW2, rb2):
    b, l, d = x.shape
    xt = x.reshape(b * l, d)
    i1, i2, g1, g2 = _router(xt, Wr, br)
    shared = _shared(xt, sW1, sb1, sW2, sb2)
    eflat = jnp.concatenate([i1.reshape(T), i2.reshape(T)])
    gflat = jnp.concatenate([g1.reshape(T), g2.reshape(T)])
    xd, gd, dest, meta = _dispatch(eflat, gflat, xt)
    eo = _ffn(meta, xd, gd, rW1, rb1, rW2, rb2)
    out = _combine(dest, eo, shared)
    return out.reshape(b, l, d)
